# baseline (device time: 45813 ns/iter reference)
import jax
import jax.numpy as jnp
from jax import lax
from jax.experimental import pallas as pl
from jax.experimental.pallas import tpu as pltpu

N_DEV = 4
N_HOPS = N_DEV - 1
N_STREAMS = 2


def kernel(x):
    m, n = x.shape
    chunk = m // N_DEV
    sub = chunk // N_STREAMS
    nh = n // 2

    def body(x_ref, out_ref, acc_ref, ag_ref, rsr_ref, rsl_ref,
             send_sems, recv_sems):
        p = lax.axis_index("i")
        left = (p + N_DEV - 1) % N_DEV
        right = (p + 1) % N_DEV

        def sem(phase, d, k, s):
            return ((phase * 2 + d) * N_STREAMS + k) * N_HOPS + s

        def rs_send_off(d, s):
            c = (p + N_DEV - s) % N_DEV if d == 0 else (p + s) % N_DEV
            return c * chunk

        def rs_recv_off(d, s):
            c = (p + N_DEV - 1 - s) % N_DEV if d == 0 else (p + s + 1) % N_DEV
            return c * chunk

        def own_off(d):
            c = (p + 1) % N_DEV if d == 0 else (p + N_DEV - 1) % N_DEV
            return c * chunk

        def ag_off(d, s):
            c = (p + 1 + N_DEV - s) % N_DEV if d == 0 else (p + N_DEV - 1 + s) % N_DEV
            return c * chunk

        def col(d):
            return pl.ds(0, nh) if d == 0 else pl.ds(nh, nh)

        def nbr(d):
            return (right,) if d == 0 else (left,)

        rs_recv_ref = (rsr_ref, rsl_ref)
        all_rdmas = []

        def start_rs(d, k, s):
            rdma = pltpu.make_async_remote_copy(
                src_ref=acc_ref.at[d, pl.ds(rs_send_off(d, s) + k * sub, sub), :],
                dst_ref=rs_recv_ref[d].at[s, pl.ds(k * sub, sub), :],
                send_sem=send_sems.at[sem(0, d, k, s)],
                recv_sem=recv_sems.at[sem(0, d, k, s)],
                device_id=nbr(d),
                device_id_type=pl.DeviceIdType.MESH,
            )
            rdma.start()
            all_rdmas.append(rdma)
            return rdma

        def start_ag(d, k, s):
            rows = pl.ds(ag_off(d, s) + k * sub, sub)
            src = acc_ref if s == 0 else ag_ref
            rdma = pltpu.make_async_remote_copy(
                src_ref=src.at[d, rows, :],
                dst_ref=ag_ref.at[d, rows, :],
                send_sem=send_sems.at[sem(1, d, k, s)],
                recv_sem=recv_sems.at[sem(1, d, k, s)],
                device_id=nbr(d),
                device_id_type=pl.DeviceIdType.MESH,
            )
            rdma.start()
            all_rdmas.append(rdma)
            return rdma

        barrier_sem = pltpu.get_barrier_semaphore()
        for d in range(2):
            pl.semaphore_signal(
                barrier_sem, inc=1,
                device_id=nbr(d), device_id_type=pl.DeviceIdType.MESH,
            )
        pl.semaphore_wait(barrier_sem, 2)

        prow = pl.ds(p * chunk, chunk)
        for d in range(2):
            acc_ref[d, prow, :] = x_ref[prow, col(d)].astype(jnp.bfloat16)
        rs_rdmas = {}
        for d in range(2):
            for k in range(N_STREAMS):
                rs_rdmas[(d, k, 0)] = start_rs(d, k, 0)
        for j in range(1, N_DEV):
            row = pl.ds(((p + j) % N_DEV) * chunk, chunk)
            for d in range(2):
                acc_ref[d, row, :] = x_ref[row, col(d)].astype(jnp.bfloat16)

        ag_rdmas = {}
        for s in range(N_HOPS):
            for d in range(2):
                for k in range(N_STREAMS):
                    rs_rdmas[(d, k, s)].wait_recv()
                    rows = pl.ds(rs_recv_off(d, s) + k * sub, sub)
                    acc_ref[d, rows, :] += rs_recv_ref[d][s, pl.ds(k * sub, sub), :]
                    if s < N_HOPS - 1:
                        rs_rdmas[(d, k, s + 1)] = start_rs(d, k, s + 1)
                    else:
                        ag_rdmas[(d, k, 0)] = start_ag(d, k, 0)
                        own_rows = pl.ds(own_off(d) + k * sub, sub)
                        out_ref[own_rows, col(d)] = acc_ref[d, own_rows, :]

        for s in range(N_HOPS):
            for d in range(2):
                for k in range(N_STREAMS):
                    ag_rdmas[(d, k, s)].wait_recv()
                    if s < N_HOPS - 1:
                        ag_rdmas[(d, k, s + 1)] = start_ag(d, k, s + 1)
                    rows = pl.ds(ag_off(d, s) + k * sub, sub)
                    out_ref[rows, col(d)] = ag_ref[d, rows, :]

        for rdma in all_rdmas:
            rdma.wait_send()

    return pl.pallas_call(
        body,
        out_shape=jax.ShapeDtypeStruct((m, n), jnp.bfloat16),
        in_specs=[pl.BlockSpec(memory_space=pltpu.VMEM)],
        out_specs=pl.BlockSpec(memory_space=pltpu.VMEM),
        scratch_shapes=[
            pltpu.VMEM((2, m, nh), jnp.bfloat16),
            pltpu.VMEM((2, m, nh), jnp.bfloat16),
            pltpu.VMEM((N_HOPS, chunk, nh), jnp.bfloat16),
            pltpu.VMEM((N_HOPS, chunk, nh), jnp.bfloat16),
            pltpu.SemaphoreType.DMA((2 * 2 * N_STREAMS * N_HOPS,)),
            pltpu.SemaphoreType.DMA((2 * 2 * N_STREAMS * N_HOPS,)),
        ],
        compiler_params=pltpu.CompilerParams(collective_id=0),
    )(x)


# device time: 45767 ns/iter; 1.0010x vs baseline; 1.0010x over previous
import jax
import jax.numpy as jnp
from jax import lax
from jax.experimental import pallas as pl
from jax.experimental.pallas import tpu as pltpu

N_DEV = 4
N_HOPS = N_DEV - 1
N_STREAMS = 2


def kernel(x):
    m, n = x.shape
    chunk = m // N_DEV
    sub = chunk // N_STREAMS
    nh = n // 2

    def body(x_ref, out_ref, acc_ref, ag_ref, rsr_ref, rsl_ref,
             send_sems, recv_sems):
        p = lax.axis_index("i")
        left = (p + N_DEV - 1) % N_DEV
        right = (p + 1) % N_DEV

        def sem(phase, d, k, s):
            return ((phase * 2 + d) * N_STREAMS + k) * N_HOPS + s

        def rs_send_off(d, s):
            c = (p + N_DEV - s) % N_DEV if d == 0 else (p + s) % N_DEV
            return c * chunk

        def rs_recv_off(d, s):
            c = (p + N_DEV - 1 - s) % N_DEV if d == 0 else (p + s + 1) % N_DEV
            return c * chunk

        def own_off(d):
            c = (p + 1) % N_DEV if d == 0 else (p + N_DEV - 1) % N_DEV
            return c * chunk

        def ag_off(d, s):
            c = (p + 1 + N_DEV - s) % N_DEV if d == 0 else (p + N_DEV - 1 + s) % N_DEV
            return c * chunk

        def ag_recv_off(d, s):
            c = (p + N_DEV - s) % N_DEV if d == 0 else (p + s) % N_DEV
            return c * chunk

        def col(d):
            return pl.ds(0, nh) if d == 0 else pl.ds(nh, nh)

        def nbr(d):
            return (right,) if d == 0 else (left,)

        rs_recv_ref = (rsr_ref, rsl_ref)
        all_rdmas = []

        def start_rs(d, k, s):
            rdma = pltpu.make_async_remote_copy(
                src_ref=acc_ref.at[d, pl.ds(rs_send_off(d, s) + k * sub, sub), :],
                dst_ref=rs_recv_ref[d].at[s, pl.ds(k * sub, sub), :],
                send_sem=send_sems.at[sem(0, d, k, s)],
                recv_sem=recv_sems.at[sem(0, d, k, s)],
                device_id=nbr(d),
                device_id_type=pl.DeviceIdType.MESH,
            )
            rdma.start()
            all_rdmas.append(rdma)
            return rdma

        def start_ag(d, k, s):
            rows = pl.ds(ag_off(d, s) + k * sub, sub)
            src = acc_ref if s == 0 else ag_ref
            rdma = pltpu.make_async_remote_copy(
                src_ref=src.at[d, rows, :],
                dst_ref=ag_ref.at[d, rows, :],
                send_sem=send_sems.at[sem(1, d, k, s)],
                recv_sem=recv_sems.at[sem(1, d, k, s)],
                device_id=nbr(d),
                device_id_type=pl.DeviceIdType.MESH,
            )
            rdma.start()
            all_rdmas.append(rdma)
            return rdma

        barrier_sem = pltpu.get_barrier_semaphore()
        for d in range(2):
            pl.semaphore_signal(
                barrier_sem, inc=1,
                device_id=nbr(d), device_id_type=pl.DeviceIdType.MESH,
            )
        pl.semaphore_wait(barrier_sem, 2)

        prow = pl.ds(p * chunk, chunk)
        for d in range(2):
            acc_ref[d, prow, :] = x_ref[prow, col(d)].astype(jnp.bfloat16)
        rs_rdmas = {}
        for d in range(2):
            for k in range(N_STREAMS):
                rs_rdmas[(d, k, 0)] = start_rs(d, k, 0)
        for j in range(1, N_DEV):
            row = pl.ds(((p + j) % N_DEV) * chunk, chunk)
            for d in range(2):
                acc_ref[d, row, :] = x_ref[row, col(d)].astype(jnp.bfloat16)

        ag_rdmas = {}
        for s in range(N_HOPS):
            for d in range(2):
                for k in range(N_STREAMS):
                    rs_rdmas[(d, k, s)].wait_recv()
                    rows = pl.ds(rs_recv_off(d, s) + k * sub, sub)
                    acc_ref[d, rows, :] += rs_recv_ref[d][s, pl.ds(k * sub, sub), :]
                    if s < N_HOPS - 1:
                        rs_rdmas[(d, k, s + 1)] = start_rs(d, k, s + 1)
                    else:
                        ag_rdmas[(d, k, 0)] = start_ag(d, k, 0)
                        own_rows = pl.ds(own_off(d) + k * sub, sub)
                        out_ref[own_rows, col(d)] = acc_ref[d, own_rows, :]

        for s in range(N_HOPS):
            for d in range(2):
                for k in range(N_STREAMS):
                    ag_rdmas[(d, k, s)].wait_recv()
                    if s < N_HOPS - 1:
                        ag_rdmas[(d, k, s + 1)] = start_ag(d, k, s + 1)
                    rows = pl.ds(ag_recv_off(d, s) + k * sub, sub)
                    out_ref[rows, col(d)] = ag_ref[d, rows, :]

        for rdma in all_rdmas:
            rdma.wait_send()

    return pl.pallas_call(
        body,
        out_shape=jax.ShapeDtypeStruct((m, n), jnp.bfloat16),
        in_specs=[pl.BlockSpec(memory_space=pltpu.VMEM)],
        out_specs=pl.BlockSpec(memory_space=pltpu.VMEM),
        scratch_shapes=[
            pltpu.VMEM((2, m, nh), jnp.bfloat16),
            pltpu.VMEM((2, m, nh), jnp.bfloat16),
            pltpu.VMEM((N_HOPS, chunk, nh), jnp.bfloat16),
            pltpu.VMEM((N_HOPS, chunk, nh), jnp.bfloat16),
            pltpu.SemaphoreType.DMA((2 * 2 * N_STREAMS * N_HOPS,)),
            pltpu.SemaphoreType.DMA((2 * 2 * N_STREAMS * N_HOPS,)),
        ],
        compiler_params=pltpu.CompilerParams(collective_id=0),
    )(x)


# device time: 44491 ns/iter; 1.0297x vs baseline; 1.0287x over previous
import jax
import jax.numpy as jnp
from jax import lax
from jax.experimental import pallas as pl
from jax.experimental.pallas import tpu as pltpu

N_DEV = 4


def kernel(x):
    m, n = x.shape
    half = m // 2
    quar = m // 4
    subq = quar // 2
    nh = n // 2

    def body(x_ref, out_ref, acc_ref, s1r_ref, s2r_ref, send_sems, recv_sems):
        p = lax.axis_index("i")
        lab = p ^ (p >> 1)
        b_y = lab & 1
        b_x = (lab >> 1) & 1
        g1 = lab ^ 1
        g2 = lab ^ 2
        q1 = g1 ^ (g1 >> 1)
        q2 = g2 ^ (g2 >> 1)

        def qf(d):
            return (q1,) if d == 0 else (q2,)

        def qs(d):
            return (q2,) if d == 0 else (q1,)

        def b1(d):
            return b_y if d == 0 else b_x

        def b2(d):
            return b_x if d == 0 else b_y

        def col(d):
            return pl.ds(0, nh) if d == 0 else pl.ds(nh, nh)

        def piece_rows(d, keep, j):
            base = (b1(d) if keep else 1 - b1(d)) * half
            qoff = ((1 - b2(d)) if j < 2 else b2(d)) * quar
            return base + qoff + (j % 2) * subq

        def sem(d, i):
            return d * 12 + i

        all_rdmas = []

        def start(src_rows, dst_ref, dst_rows, d, i, partner):
            rdma = pltpu.make_async_remote_copy(
                src_ref=acc_ref.at[d, pl.ds(src_rows, subq), :],
                dst_ref=dst_ref.at[d, pl.ds(dst_rows, subq), :],
                send_sem=send_sems.at[sem(d, i)],
                recv_sem=recv_sems.at[sem(d, i)],
                device_id=partner,
                device_id_type=pl.DeviceIdType.MESH,
            )
            rdma.start()
            all_rdmas.append(rdma)
            return rdma

        barrier_sem = pltpu.get_barrier_semaphore()
        for nbr in (q1, q2):
            pl.semaphore_signal(
                barrier_sem, inc=1,
                device_id=(nbr,), device_id_type=pl.DeviceIdType.MESH,
            )
        pl.semaphore_wait(barrier_sem, 2)

        s1 = {}
        for j in range(4):
            for d in range(2):
                rows = piece_rows(d, False, j)
                acc_ref[d, pl.ds(rows, subq), :] = x_ref[
                    pl.ds(rows, subq), col(d)
                ].astype(jnp.bfloat16)
                s1[(d, j)] = start(rows, s1r_ref, j * subq, d, j, qf(d))
        for j in range(4):
            for d in range(2):
                rows = piece_rows(d, True, j)
                acc_ref[d, pl.ds(rows, subq), :] = x_ref[
                    pl.ds(rows, subq), col(d)
                ].astype(jnp.bfloat16)

        s2 = {}
        for j in range(4):
            for d in range(2):
                s1[(d, j)].wait_recv()
                rows = piece_rows(d, True, j)
                acc_ref[d, pl.ds(rows, subq), :] += s1r_ref[
                    d, pl.ds(j * subq, subq), :
                ]
                if j < 2:
                    s2[(d, j)] = start(rows, s2r_ref, j * subq, d, 4 + j, qs(d))

        s3 = {}
        s4 = {}
        for i in range(2):
            for d in range(2):
                s2[(d, i)].wait_recv()
                own_rows = b1(d) * half + b2(d) * quar + i * subq
                acc_ref[d, pl.ds(own_rows, subq), :] += s2r_ref[
                    d, pl.ds(i * subq, subq), :
                ]
                s3[(d, i)] = start(own_rows, acc_ref, own_rows, d, 6 + i, qs(d))
                s4[(d, 2 + i)] = start(own_rows, acc_ref, own_rows, d,
                                       8 + 2 + i, qf(d))
                out_ref[pl.ds(own_rows, subq), col(d)] = acc_ref[
                    d, pl.ds(own_rows, subq), :
                ]

        for i in range(2):
            for d in range(2):
                s3[(d, i)].wait_recv()
                rows = b1(d) * half + (1 - b2(d)) * quar + i * subq
                s4[(d, i)] = start(rows, acc_ref, rows, d, 8 + i, qf(d))
                out_ref[pl.ds(rows, subq), col(d)] = acc_ref[
                    d, pl.ds(rows, subq), :
                ]

        for j in range(4):
            for d in range(2):
                s4[(d, j)].wait_recv()
                rows = piece_rows(d, False, j)
                out_ref[pl.ds(rows, subq), col(d)] = acc_ref[
                    d, pl.ds(rows, subq), :
                ]

        for rdma in all_rdmas:
            rdma.wait_send()

    return pl.pallas_call(
        body,
        out_shape=jax.ShapeDtypeStruct((m, n), jnp.bfloat16),
        in_specs=[pl.BlockSpec(memory_space=pltpu.VMEM)],
        out_specs=pl.BlockSpec(memory_space=pltpu.VMEM),
        scratch_shapes=[
            pltpu.VMEM((2, m, nh), jnp.bfloat16),
            pltpu.VMEM((2, half, nh), jnp.bfloat16),
            pltpu.VMEM((2, quar, nh), jnp.bfloat16),
            pltpu.SemaphoreType.DMA((24,)),
            pltpu.SemaphoreType.DMA((24,)),
        ],
        compiler_params=pltpu.CompilerParams(collective_id=0),
    )(x)
